# Initial kernel scaffold; baseline (speedup 1.0000x reference)
import jax
import jax.numpy as jnp
from jax.experimental import pallas as pl


def kernel(x, edge_index, W0, b0, W1, b1, W2, b2):
    raise NotImplementedError("write your pallas kernel here")


# speedup vs baseline: 13.4515x; 13.4515x over previous
import functools

import jax
import jax.numpy as jnp
from jax import lax
from jax.experimental import pallas as pl
from jax.experimental.pallas import tpu as pltpu
from jax.experimental.pallas import tpu_sc as plsc

N = 10000
D = 128
E = 320000
CHUNK = 128
NCHUNK = E // CHUNK
NC = 2
NS = 16
NW = NC * NS
FULL_ROUNDS = NCHUNK // NW
TAIL = NCHUNK - FULL_ROUNDS * NW
NP = 10240
RPT = NP // NS
ZB = 16
NZ = RPT // ZB

_mesh = plsc.VectorSubcoreMesh(core_axis_name="c", subcore_axis_name="s")
_f32 = jnp.float32


@functools.partial(
    pl.kernel,
    out_type=jax.ShapeDtypeStruct((NC * NP,), _f32),
    mesh=_mesh,
    scratch_types=[
        pltpu.VMEM((CHUNK,), jnp.int32),
        pltpu.VMEM((CHUNK,), _f32),
        pltpu.VMEM((RPT,), _f32),
        pltpu.VMEM_SHARED((NP,), _f32),
    ],
)
def _sc_deg(dst_hbm, out_hbm, dstb, ones, stg, acc):
    c = lax.axis_index("c")
    s = lax.axis_index("s")
    wid = s * NC + c
    for i in range(CHUNK // 16):
        ones[pl.ds(i * 16, 16)] = jnp.full((16,), 1.0, _f32)
    for i in range(RPT // 16):
        stg[pl.ds(i * 16, 16)] = jnp.zeros((16,), _f32)
    pltpu.sync_copy(stg, acc.at[pl.ds(s * RPT, RPT)])
    plsc.subcore_barrier()

    def step(cid):
        pltpu.sync_copy(dst_hbm.at[cid], dstb)
        pltpu.sync_copy(ones, acc.at[dstb], add=True)

    def body(j, carry):
        step(wid * FULL_ROUNDS + j)
        return carry

    lax.fori_loop(0, FULL_ROUNDS, body, 0)

    @pl.when(wid < TAIL)
    def _():
        step(FULL_ROUNDS * NW + wid)

    plsc.subcore_barrier()
    pltpu.sync_copy(acc.at[pl.ds(s * RPT, RPT)], stg)
    pltpu.sync_copy(stg, out_hbm.at[pl.ds(c * NP + s * RPT, RPT)])


@functools.partial(
    pl.kernel,
    out_type=jax.ShapeDtypeStruct((NC, NP, D), _f32),
    mesh=_mesh,
    scratch_types=[
        pltpu.VMEM((CHUNK,), jnp.int32),
        pltpu.VMEM((CHUNK,), jnp.int32),
        pltpu.VMEM((CHUNK, D), _f32),
        pltpu.VMEM((ZB, D), _f32),
        pltpu.VMEM_SHARED((NP, D), _f32),
        pltpu.SemaphoreType.DMA,
    ],
)
def _sc_scatter(g_hbm, src_hbm, dst_hbm, out_hbm,
                srcb, dstb, rows, stg, acc, sem):
    c = lax.axis_index("c")
    s = lax.axis_index("s")
    wid = s * NC + c
    for i in range(ZB):
        for k in range(D // 16):
            stg[i, pl.ds(k * 16, 16)] = jnp.zeros((16,), _f32)

    def zinit(j, carry):
        pltpu.sync_copy(stg, acc.at[pl.ds(s * RPT + j * ZB, ZB)])
        return carry

    lax.fori_loop(0, NZ, zinit, 0)
    plsc.subcore_barrier()

    def step(cid):
        pltpu.sync_copy(src_hbm.at[cid], srcb)
        pltpu.sync_copy(dst_hbm.at[cid], dstb)
        pltpu.async_copy(g_hbm.at[srcb], rows, sem).wait()
        pltpu.sync_copy(rows, acc.at[dstb], add=True)

    def body(j, carry):
        step(wid * FULL_ROUNDS + j)
        return carry

    lax.fori_loop(0, FULL_ROUNDS, body, 0)

    @pl.when(wid < TAIL)
    def _():
        step(FULL_ROUNDS * NW + wid)

    plsc.subcore_barrier()

    def drain(j, carry):
        base = s * RPT + j * ZB
        pltpu.sync_copy(acc.at[pl.ds(base, ZB)], stg)
        pltpu.sync_copy(stg, out_hbm.at[c, pl.ds(base, ZB)])
        return carry

    lax.fori_loop(0, NZ, drain, 0)


def _tc_first(x_ref, w_ref, deg_ref, g_ref, dinv_ref):
    dsum = deg_ref[0] + deg_ref[1] + 1.0
    dinv_bc = jnp.broadcast_to(lax.rsqrt(dsum), (N, D))
    h = jnp.dot(x_ref[...], w_ref[...],
                preferred_element_type=_f32,
                precision=lax.Precision.HIGHEST)
    g_ref[...] = h * dinv_bc
    dinv_ref[...] = dinv_bc


def _tc_mid(s_ref, g_ref, dinv_ref, b_ref, w_ref, o_ref):
    t = (s_ref[0] + s_ref[1] + g_ref[...]) * dinv_ref[...] + b_ref[...]
    t = jnp.maximum(t, 0.0)
    t = t - jnp.mean(t, axis=0, keepdims=True)
    t = t * lax.rsqrt(1e-5 + jnp.sum(t * t) / N)
    h = jnp.dot(t, w_ref[...],
                preferred_element_type=_f32,
                precision=lax.Precision.HIGHEST)
    o_ref[...] = h * dinv_ref[...]


def _tc_last(s_ref, g_ref, dinv_ref, b_ref, o_ref):
    t = (s_ref[0] + s_ref[1] + g_ref[...]) * dinv_ref[...] + b_ref[...]
    o_ref[...] = jnp.maximum(t, 0.0)


_tc_first_call = pl.pallas_call(
    _tc_first,
    out_shape=[jax.ShapeDtypeStruct((N, D), _f32),
               jax.ShapeDtypeStruct((N, D), _f32)],
)
_tc_mid_call = pl.pallas_call(
    _tc_mid, out_shape=jax.ShapeDtypeStruct((N, D), _f32))
_tc_last_call = pl.pallas_call(
    _tc_last, out_shape=jax.ShapeDtypeStruct((N, D), _f32))


def kernel(x, edge_index, W0, b0, W1, b1, W2, b2):
    src2 = edge_index[0].reshape(NCHUNK, CHUNK)
    dst2 = edge_index[1].reshape(NCHUNK, CHUNK)

    degp = _sc_deg(dst2).reshape(NC, NP)[:, :N]
    deg3 = degp.reshape(NC, N, 1)
    g0, dinv_bc = _tc_first_call(x, W0, deg3)
    s = _sc_scatter(g0, src2, dst2)[:, :N]
    g1 = _tc_mid_call(s, g0, dinv_bc, b0.reshape(1, D), W1)
    s = _sc_scatter(g1, src2, dst2)[:, :N]
    g2 = _tc_mid_call(s, g1, dinv_bc, b1.reshape(1, D), W2)
    s = _sc_scatter(g2, src2, dst2)[:, :N]
    return _tc_last_call(s, g2, dinv_bc, b2.reshape(1, D))
